# Initial kernel scaffold; baseline (speedup 1.0000x reference)
import jax
import jax.numpy as jnp
from jax.experimental import pallas as pl


def kernel(x, edge_index, params):
    raise NotImplementedError("write your pallas kernel here")


# speedup vs baseline: 31.0579x; 31.0579x over previous
import functools

import jax
import jax.numpy as jnp
from jax import lax
from jax.experimental import pallas as pl
from jax.experimental.pallas import tpu as pltpu
from jax.experimental.pallas import tpu_sc as plsc

N = 10000
FEA = 128
E = 320000
EL = E + N
NC, NS = 2, 16
NW = NC * NS
CH = 128
NCHUNK = -(-EL // (NW * CH))
EP = NW * CH * NCHUNK
RPW = 640
NR = RPW * NS
ZB = RPW // 4

_PREC = jax.lax.Precision.HIGHEST
_NEG = -3e38


def _dot(a, b):
    return jax.lax.dot(a, b, precision=_PREC,
                       preferred_element_type=jnp.float32)


def _elu(z):
    return jnp.where(z > 0, z, jnp.exp(jnp.minimum(z, 0.0)) - 1.0)


def _tc1_body(*refs):
    x_ref = refs[0]
    w_refs = refs[1:4]
    as_refs = refs[4:7]
    ad_refs = refs[7:10]
    h_refs = refs[10:13]
    es_ref, ed_ref, mes_ref, med_ref = refs[13:]
    i = pl.program_id(0)
    x = x_ref[...]
    es = jnp.zeros((x.shape[0], 16), jnp.float32)
    ed = jnp.zeros((x.shape[0], 16), jnp.float32)
    for k in range(3):
        h = _dot(x, w_refs[k][...])
        h_refs[k][...] = h
        es = es + _dot(h, as_refs[k][...])
        ed = ed + _dot(h, ad_refs[k][...])
    es_ref[...] = es
    ed_ref[...] = ed

    @pl.when(i == 0)
    def _():
        mes_ref[...] = jnp.full((8, 16), _NEG, jnp.float32)
        med_ref[...] = jnp.full((8, 16), _NEG, jnp.float32)

    mes_ref[...] = jnp.maximum(mes_ref[...],
                               jnp.broadcast_to(jnp.max(es, axis=0), (8, 16)))
    med_ref[...] = jnp.maximum(med_ref[...],
                               jnp.broadcast_to(jnp.max(ed, axis=0), (8, 16)))


def _tc_stage1(xp, Wks, Asks, Adks):
    BLK = 2560
    G = NR // BLK
    const = lambda a: pl.BlockSpec(a.shape, lambda i: (0, 0))
    row = lambda w: pl.BlockSpec((BLK, w), lambda i: (i, 0))
    return pl.pallas_call(
        _tc1_body,
        grid=(G,),
        in_specs=[row(FEA)] + [const(a) for a in Wks + Asks + Adks],
        out_specs=[row(128), row(128), row(128), row(16), row(16),
                   const(jnp.zeros((8, 16))), const(jnp.zeros((8, 16)))],
        out_shape=[jax.ShapeDtypeStruct((NR, 128), jnp.float32)] * 3 +
                  [jax.ShapeDtypeStruct((NR, 16), jnp.float32)] * 2 +
                  [jax.ShapeDtypeStruct((8, 16), jnp.float32)] * 2,
    )(xp, *Wks, *Asks, *Adks)


def _tc2_body(*refs):
    p0s = refs[0:3]
    p1s = refs[3:6]
    bs = refs[6:9]
    w4s = refs[9:12]
    as_ref, ad_ref = refs[12:14]
    h_ref, es_ref, ed_ref, mes_ref, med_ref = refs[14:]
    i = pl.program_id(0)
    h = jnp.zeros((p0s[0].shape[0], 128), jnp.float32)
    for k in range(3):
        z = _elu(p0s[k][...] + p1s[k][...] + bs[k][...])
        h = h + _dot(z, w4s[k][...])
    h_ref[...] = h
    es = _dot(h, as_ref[...])
    ed = _dot(h, ad_ref[...])
    es_ref[...] = es
    ed_ref[...] = ed

    @pl.when(i == 0)
    def _():
        mes_ref[...] = jnp.full((8, 16), _NEG, jnp.float32)
        med_ref[...] = jnp.full((8, 16), _NEG, jnp.float32)

    mes_ref[...] = jnp.maximum(mes_ref[...],
                               jnp.broadcast_to(jnp.max(es, axis=0), (8, 16)))
    med_ref[...] = jnp.maximum(med_ref[...],
                               jnp.broadcast_to(jnp.max(ed, axis=0), (8, 16)))


def _tc_stage2(p0s, p1s, bks, W4ks, As4, Ad4):
    BLK = 2560
    G = NR // BLK
    const = lambda a: pl.BlockSpec(a.shape, lambda i: (0, 0))
    row = lambda w: pl.BlockSpec((BLK, w), lambda i: (i, 0))
    in_specs = ([row(48)] * 6 + [const(b) for b in bks]
                + [const(w) for w in W4ks] + [const(As4), const(Ad4)])
    return pl.pallas_call(
        _tc2_body,
        grid=(G,),
        in_specs=in_specs,
        out_specs=[row(128), row(16), row(16),
                   const(jnp.zeros((8, 16))), const(jnp.zeros((8, 16)))],
        out_shape=[jax.ShapeDtypeStruct((NR, 128), jnp.float32),
                   jax.ShapeDtypeStruct((NR, 16), jnp.float32),
                   jax.ShapeDtypeStruct((NR, 16), jnp.float32),
                   jax.ShapeDtypeStruct((8, 16), jnp.float32),
                   jax.ShapeDtypeStruct((8, 16), jnp.float32)],
    )(*p0s, *p1s, *bks, *W4ks, As4, Ad4)


def _tc3_body(*refs):
    p0_ref, p1_ref, b_ref = refs[0], refs[1], refs[2]
    w_refs = refs[3:-1]
    out_ref = refs[-1]
    z = _elu(p0_ref[...] + p1_ref[...] + b_ref[...])
    for li in range(len(w_refs) // 4):
        W, b, s, bt = w_refs[4 * li:4 * li + 4]
        z = jnp.maximum(_dot(z, W[...]) + b[...], 0.0) * s[...] + bt[...]
    out_ref[...] = z


def _tc_stage3(p0, p1, b, layers):
    BLK = 2560
    G = NR // BLK
    Fin = p0.shape[1]
    flat = []
    for lay in layers:
        flat.extend(lay)
    in_specs = [pl.BlockSpec((BLK, Fin), lambda i: (i, 0)),
                pl.BlockSpec((BLK, Fin), lambda i: (i, 0)),
                pl.BlockSpec((1, Fin), lambda i: (0, 0))]
    for a in flat:
        in_specs.append(pl.BlockSpec(a.shape, lambda i: (0, 0)))
    Fout = layers[-1][0].shape[1]
    return pl.pallas_call(
        _tc3_body,
        grid=(G,),
        in_specs=in_specs,
        out_specs=pl.BlockSpec((BLK, Fout), lambda i: (i, 0)),
        out_shape=jax.ShapeDtypeStruct((NR, Fout), jnp.float32),
    )(p0, p1, b, *flat)


_MESH = functools.partial(plsc.VectorSubcoreMesh,
                          core_axis_name="c", subcore_axis_name="s",
                          num_cores=NC, num_subcores=NS)
_SC_PARAMS = functools.partial(pltpu.CompilerParams,
                               use_tc_tiling_on_sc=False)


def _sc_pass1(es, ed, mes, med, srcp, dstp):
    @functools.partial(
        pl.kernel,
        out_type=[jax.ShapeDtypeStruct((EP, 16), jnp.float32),
                  jax.ShapeDtypeStruct((NC, NR, 16), jnp.float32)],
        mesh=_MESH(),
        compiler_params=_SC_PARAMS(),
        scratch_types=[
            pltpu.VMEM((CH,), jnp.int32),
            pltpu.VMEM((CH,), jnp.int32),
            pltpu.VMEM((CH, 16), jnp.float32),
            pltpu.VMEM((CH, 16), jnp.float32),
            pltpu.VMEM((CH, 16), jnp.float32),
            pltpu.VMEM((8, 16), jnp.float32),
            pltpu.VMEM((8, 16), jnp.float32),
            pltpu.VMEM((RPW, 16), jnp.float32),
            pltpu.VMEM_SHARED((NR, 16), jnp.float32),
            pltpu.VMEM_SHARED((NR, 16), jnp.float32),
            pltpu.VMEM_SHARED((NR, 16), jnp.float32),
            pltpu.SemaphoreType.DMA,
        ],
    )
    def k(es_hbm, ed_hbm, mes_hbm, med_hbm, src_hbm, dst_hbm,
          ee_hbm, denp_hbm,
          src_v, dst_v, esg, edg, eev, mbufa, mbufb, zbuf,
          den_sh, es_sh, ed_sh, sem):
        cid = lax.axis_index("c")
        sid = lax.axis_index("s")
        wid = sid * NC + cid
        r0 = sid * RPW

        pltpu.sync_copy(mes_hbm, mbufa)
        pltpu.sync_copy(med_hbm, mbufb)
        msum = mbufa[0] + mbufb[0]
        mr = jnp.maximum(msum, 0.2 * msum)

        pltpu.sync_copy(es_hbm.at[pl.ds(r0, RPW)], zbuf)
        pltpu.sync_copy(zbuf, es_sh.at[pl.ds(r0, RPW)])
        pltpu.sync_copy(ed_hbm.at[pl.ds(r0, RPW)], zbuf)
        pltpu.sync_copy(zbuf, ed_sh.at[pl.ds(r0, RPW)])

        z16 = jnp.zeros((16,), jnp.float32)

        def zb(i, c):
            zbuf[i] = z16
            return c

        lax.fori_loop(0, RPW, zb, 0)
        pltpu.sync_copy(zbuf, den_sh.at[pl.ds(r0, RPW)])
        plsc.subcore_barrier()

        def chunk(kc, m):
            base = (wid * NCHUNK + kc) * CH
            pltpu.sync_copy(src_hbm.at[pl.ds(base, CH)], src_v)
            pltpu.sync_copy(dst_hbm.at[pl.ds(base, CH)], dst_v)
            pltpu.async_copy(es_sh.at[src_v], esg, sem).wait()
            pltpu.async_copy(ed_sh.at[dst_v], edg, sem).wait()

            def eb(i, mm):
                e = esg[i] + edg[i]
                e = jnp.maximum(e, 0.2 * e)
                eev[i] = jnp.exp(e - mm)
                return mm

            m = lax.fori_loop(0, CH, eb, m)
            pltpu.sync_copy(eev, ee_hbm.at[pl.ds(base, CH)])
            pltpu.sync_copy(eev, den_sh.at[dst_v], add=True)
            return m

        lax.fori_loop(0, NCHUNK, chunk, mr)
        plsc.subcore_barrier()
        pltpu.sync_copy(den_sh.at[pl.ds(r0, RPW)],
                        denp_hbm.at[cid, pl.ds(r0, RPW)])

    return k(es, ed, mes, med, srcp, dstp)


def _sc_pass2(h, ee, srcp, dstp, denp, F, CHN, j0):
    NV = F // 16
    H0 = [(16 * (j0 + j)) // CHN for j in range(NV)]
    TH = [(H0[j] + 1) * CHN - 16 * (j0 + j) for j in range(NV)]
    H1 = [min(H0[j] + 1, 15) for j in range(NV)]

    @functools.partial(
        pl.kernel,
        out_type=jax.ShapeDtypeStruct((NC, NR, F), jnp.float32),
        mesh=_MESH(),
        compiler_params=_SC_PARAMS(),
        scratch_types=[
            pltpu.VMEM((CH,), jnp.int32),
            pltpu.VMEM((CH,), jnp.int32),
            pltpu.VMEM((CH, 16), jnp.float32),
            pltpu.VMEM((CH, 16), jnp.float32),
            pltpu.VMEM((CH, 16), jnp.float32),
            pltpu.VMEM((CH, 128), jnp.float32),
            pltpu.VMEM((CH, F), jnp.float32),
            pltpu.VMEM((RPW, 16), jnp.float32),
            pltpu.VMEM((RPW, 16), jnp.float32),
            pltpu.VMEM((ZB, F), jnp.float32),
            pltpu.VMEM_SHARED((NR, 16), jnp.float32),
            pltpu.VMEM_SHARED((NR, F), jnp.float32),
            pltpu.SemaphoreType.DMA,
        ],
    )
    def k(h_hbm, ee_hbm, src_hbm, dst_hbm, denp_hbm, outp_hbm,
          src_v, dst_v, eev, deng, alpha, hg, msg, pa, pb, zbuf,
          den_sh, out_sh, sem):
        cid = lax.axis_index("c")
        sid = lax.axis_index("s")
        wid = sid * NC + cid
        r0 = sid * RPW

        pltpu.sync_copy(denp_hbm.at[0, pl.ds(r0, RPW)], pa)
        pltpu.sync_copy(denp_hbm.at[1, pl.ds(r0, RPW)], pb)

        def db(i, c):
            pa[i] = pa[i] + pb[i]
            return c

        lax.fori_loop(0, RPW, db, 0)
        pltpu.sync_copy(pa, den_sh.at[pl.ds(r0, RPW)])

        z16 = jnp.zeros((16,), jnp.float32)

        def zb(i, c):
            for j in range(NV):
                zbuf[i, pl.ds(16 * j, 16)] = z16
            return c

        lax.fori_loop(0, ZB, zb, 0)
        for q in range(RPW // ZB):
            pltpu.sync_copy(zbuf, out_sh.at[pl.ds(r0 + q * ZB, ZB)])
        plsc.subcore_barrier()

        lane = lax.iota(jnp.int32, 16)
        masks = [lane < TH[j] for j in range(NV)]

        def chunk(kc, c):
            base = (wid * NCHUNK + kc) * CH
            pltpu.sync_copy(src_hbm.at[pl.ds(base, CH)], src_v)
            pltpu.sync_copy(dst_hbm.at[pl.ds(base, CH)], dst_v)
            pltpu.sync_copy(ee_hbm.at[pl.ds(base, CH)], eev)
            pltpu.async_copy(den_sh.at[dst_v], deng, sem).wait()

            def ab(i, cc):
                alpha[i] = eev[i] / (deng[i] + 1e-16)
                return cc

            lax.fori_loop(0, CH, ab, 0)
            pltpu.async_copy(h_hbm.at[src_v], hg, sem).wait()

            def mb(i, cc):
                av = alpha[i]
                for j in range(NV):
                    a0 = av[H0[j]]
                    a1 = av[H1[j]]
                    aexp = jnp.where(masks[j], a0, a1)
                    msg[i, pl.ds(16 * j, 16)] = (
                        hg[i, pl.ds(16 * j, 16)] * aexp)
                return cc

            lax.fori_loop(0, CH, mb, 0)
            pltpu.sync_copy(msg, out_sh.at[dst_v], add=True)
            return c

        lax.fori_loop(0, NCHUNK, chunk, 0)
        plsc.subcore_barrier()
        pltpu.sync_copy(out_sh.at[pl.ds(r0, RPW)],
                        outp_hbm.at[cid, pl.ds(r0, RPW)])

    return k(h, ee, srcp, dstp, denp)


def _attn_mat(a, H, C):
    A = jnp.zeros((H * C, 16), jnp.float32)
    return A.at[jnp.arange(H * C), jnp.repeat(jnp.arange(H), C)].set(
        a.reshape(-1))


def _bn_layer(W, b, g, bt):
    c = 1.0 / jnp.sqrt(jnp.float32(1.0 + 1e-5))
    return (W, b[None], (g * c)[None], bt[None])


def _pad_cols(a, n):
    return jnp.pad(a, ((0, 0), (0, n - a.shape[1])))


def _pad_rows(a, n):
    return jnp.pad(a, ((0, n - a.shape[0]), (0, 0)))


def kernel(x, edge_index, params):
    src = edge_index[0]
    dst = edge_index[1]
    loop = jnp.arange(N, dtype=src.dtype)
    padi = jnp.full((EP - EL,), N, src.dtype)
    srcp = jnp.concatenate([src, loop, padi])
    dstp = jnp.concatenate([dst, loop, padi])
    xp = jnp.pad(x, ((0, NR - N), (0, 0)))

    W1, as1, ad1, b1 = params['conv1']
    W4, as4, ad4, b4 = params['conv4']
    As1 = _attn_mat(as1, 12, 12)
    Ad1 = _attn_mat(ad1, 12, 12)
    W1ks = [_pad_cols(W1[:, 48 * k:48 * k + 48], 128) for k in range(3)]
    As1ks = [_pad_rows(As1[48 * k:48 * k + 48], 128) for k in range(3)]
    Ad1ks = [_pad_rows(Ad1[48 * k:48 * k + 48], 128) for k in range(3)]
    b1ks = [b1[None, 48 * k:48 * k + 48] for k in range(3)]
    W4ks = [_pad_cols(W4[48 * k:48 * k + 48], 128) for k in range(3)]
    As4 = _pad_rows(_attn_mat(as4, 8, 8), 128)
    Ad4 = _pad_rows(_attn_mat(ad4, 8, 8), 128)

    layers = [_bn_layer(*lay) for lay in params['mlp']]
    heads = params['heads']
    for li in range(3):
        Ws = [h[li][0] for h in heads]
        bs = jnp.concatenate([h[li][1] for h in heads])
        gs = jnp.concatenate([h[li][2] for h in heads])
        bts = jnp.concatenate([h[li][3] for h in heads])
        Wcat = jax.scipy.linalg.block_diag(*Ws) if li > 0 else \
            jnp.concatenate(Ws, axis=1)
        if li == 2:
            Wcat = jnp.pad(Wcat, ((0, 0), (0, 5)))
            bs = jnp.pad(bs, (0, 5))
            gs = jnp.pad(gs, (0, 5))
            bts = jnp.pad(bts, (0, 5))
        layers.append(_bn_layer(Wcat, bs, gs, bts))

    h1a, h1b, h1c, es1, ed1, mes1, med1 = _tc_stage1(xp, W1ks, As1ks, Ad1ks)
    ee1, denp1 = _sc_pass1(es1, ed1, mes1, med1, srcp, dstp)
    outs1 = [_sc_pass2(hk, ee1, srcp, dstp, denp1, 48, 12, 3 * k)
             for k, hk in enumerate((h1a, h1b, h1c))]
    h2, es2, ed2, mes2, med2 = _tc_stage2(
        [o[0] for o in outs1], [o[1] for o in outs1], b1ks, W4ks, As4, Ad4)
    ee2, denp2 = _sc_pass1(es2, ed2, mes2, med2, srcp, dstp)
    outp2 = _sc_pass2(h2, ee2, srcp, dstp, denp2, 64, 8, 0)
    out = _tc_stage3(outp2[0], outp2[1], b4[None], layers)
    return out[:N, :3]


# speedup vs baseline: 32.9206x; 1.0600x over previous
import functools

import jax
import jax.numpy as jnp
from jax import lax
from jax.experimental import pallas as pl
from jax.experimental.pallas import tpu as pltpu
from jax.experimental.pallas import tpu_sc as plsc

N = 10000
FEA = 128
E = 320000
EL = E + N
NC, NS = 2, 16
NW = NC * NS
CH = 128
NCHUNK = -(-EL // (NW * CH)) + 1
EP = NW * CH * NCHUNK
RPW = 640
NR = RPW * NS
ZB = RPW // 4

_PREC = jax.lax.Precision.HIGHEST
_NEG = -3e38


def _dot(a, b):
    return jax.lax.dot(a, b, precision=_PREC,
                       preferred_element_type=jnp.float32)


def _elu(z):
    return jnp.where(z > 0, z, jnp.exp(jnp.minimum(z, 0.0)) - 1.0)


def _tc1_body(*refs):
    x_ref = refs[0]
    w_refs = refs[1:4]
    as_refs = refs[4:7]
    ad_refs = refs[7:10]
    h_refs = refs[10:13]
    es_ref, ed_ref, mes_ref, med_ref = refs[13:]
    i = pl.program_id(0)
    x = x_ref[...]
    es = jnp.zeros((x.shape[0], 16), jnp.float32)
    ed = jnp.zeros((x.shape[0], 16), jnp.float32)
    for k in range(3):
        h = _dot(x, w_refs[k][...])
        h_refs[k][...] = h
        es = es + _dot(h, as_refs[k][...])
        ed = ed + _dot(h, ad_refs[k][...])
    es_ref[...] = es
    ed_ref[...] = ed

    @pl.when(i == 0)
    def _():
        mes_ref[...] = jnp.full((8, 16), _NEG, jnp.float32)
        med_ref[...] = jnp.full((8, 16), _NEG, jnp.float32)

    mes_ref[...] = jnp.maximum(mes_ref[...],
                               jnp.broadcast_to(jnp.max(es, axis=0), (8, 16)))
    med_ref[...] = jnp.maximum(med_ref[...],
                               jnp.broadcast_to(jnp.max(ed, axis=0), (8, 16)))


def _tc_stage1(xp, Wks, Asks, Adks):
    BLK = 2560
    G = NR // BLK
    const = lambda a: pl.BlockSpec(a.shape, lambda i: (0, 0))
    row = lambda w: pl.BlockSpec((BLK, w), lambda i: (i, 0))
    return pl.pallas_call(
        _tc1_body,
        grid=(G,),
        in_specs=[row(FEA)] + [const(a) for a in Wks + Asks + Adks],
        out_specs=[row(128), row(128), row(128), row(16), row(16),
                   const(jnp.zeros((8, 16))), const(jnp.zeros((8, 16)))],
        out_shape=[jax.ShapeDtypeStruct((NR, 128), jnp.float32)] * 3 +
                  [jax.ShapeDtypeStruct((NR, 16), jnp.float32)] * 2 +
                  [jax.ShapeDtypeStruct((8, 16), jnp.float32)] * 2,
    )(xp, *Wks, *Asks, *Adks)


def _tc2_body(*refs):
    p0s = refs[0:3]
    p1s = refs[3:6]
    bs = refs[6:9]
    w4s = refs[9:12]
    as_ref, ad_ref = refs[12:14]
    h_ref, es_ref, ed_ref, mes_ref, med_ref = refs[14:]
    i = pl.program_id(0)
    h = jnp.zeros((p0s[0].shape[0], 128), jnp.float32)
    for k in range(3):
        z = _elu(p0s[k][...] + p1s[k][...] + bs[k][...])
        h = h + _dot(z, w4s[k][...])
    h_ref[...] = h
    es = _dot(h, as_ref[...])
    ed = _dot(h, ad_ref[...])
    es_ref[...] = es
    ed_ref[...] = ed

    @pl.when(i == 0)
    def _():
        mes_ref[...] = jnp.full((8, 16), _NEG, jnp.float32)
        med_ref[...] = jnp.full((8, 16), _NEG, jnp.float32)

    mes_ref[...] = jnp.maximum(mes_ref[...],
                               jnp.broadcast_to(jnp.max(es, axis=0), (8, 16)))
    med_ref[...] = jnp.maximum(med_ref[...],
                               jnp.broadcast_to(jnp.max(ed, axis=0), (8, 16)))


def _tc_stage2(p0s, p1s, bks, W4ks, As4, Ad4):
    BLK = 2560
    G = NR // BLK
    const = lambda a: pl.BlockSpec(a.shape, lambda i: (0, 0))
    row = lambda w: pl.BlockSpec((BLK, w), lambda i: (i, 0))
    in_specs = ([row(48)] * 6 + [const(b) for b in bks]
                + [const(w) for w in W4ks] + [const(As4), const(Ad4)])
    return pl.pallas_call(
        _tc2_body,
        grid=(G,),
        in_specs=in_specs,
        out_specs=[row(128), row(16), row(16),
                   const(jnp.zeros((8, 16))), const(jnp.zeros((8, 16)))],
        out_shape=[jax.ShapeDtypeStruct((NR, 128), jnp.float32),
                   jax.ShapeDtypeStruct((NR, 16), jnp.float32),
                   jax.ShapeDtypeStruct((NR, 16), jnp.float32),
                   jax.ShapeDtypeStruct((8, 16), jnp.float32),
                   jax.ShapeDtypeStruct((8, 16), jnp.float32)],
    )(*p0s, *p1s, *bks, *W4ks, As4, Ad4)


def _tcr_body(*refs):
    p0_ref, p1_ref = refs[0], refs[1]
    ne = (len(refs) - 2) // 2
    e_refs = refs[2:2 + ne]
    out_refs = refs[2 + ne:]
    rden = 1.0 / (p0_ref[...] + p1_ref[...] + 1e-16)
    for er, outr in zip(e_refs, out_refs):
        outr[...] = _dot(rden, er[...])


def _tc_rden(denp, Ereps):
    BLK = 2560
    G = NR // BLK
    const = lambda a: pl.BlockSpec(a.shape, lambda i: (0, 0))
    row = lambda w: pl.BlockSpec((BLK, w), lambda i: (i, 0))
    ne = len(Ereps)
    return pl.pallas_call(
        _tcr_body,
        grid=(G,),
        in_specs=[row(16), row(16)] + [const(e) for e in Ereps],
        out_specs=[row(128)] * ne,
        out_shape=[jax.ShapeDtypeStruct((NR, 128), jnp.float32)] * ne,
    )(denp[0], denp[1], *Ereps)


def _tc3_body(*refs):
    p0_ref, p1_ref, b_ref = refs[0], refs[1], refs[2]
    w_refs = refs[3:-1]
    out_ref = refs[-1]
    z = _elu(p0_ref[...] + p1_ref[...] + b_ref[...])
    for li in range(len(w_refs) // 4):
        W, b, s, bt = w_refs[4 * li:4 * li + 4]
        z = jnp.maximum(_dot(z, W[...]) + b[...], 0.0) * s[...] + bt[...]
    out_ref[...] = z


def _tc_stage3(p0, p1, b, layers):
    BLK = 2560
    G = NR // BLK
    Fin = p0.shape[1]
    flat = []
    for lay in layers:
        flat.extend(lay)
    in_specs = [pl.BlockSpec((BLK, Fin), lambda i: (i, 0)),
                pl.BlockSpec((BLK, Fin), lambda i: (i, 0)),
                pl.BlockSpec((1, Fin), lambda i: (0, 0))]
    for a in flat:
        in_specs.append(pl.BlockSpec(a.shape, lambda i: (0, 0)))
    Fout = layers[-1][0].shape[1]
    return pl.pallas_call(
        _tc3_body,
        grid=(G,),
        in_specs=in_specs,
        out_specs=pl.BlockSpec((BLK, Fout), lambda i: (i, 0)),
        out_shape=jax.ShapeDtypeStruct((NR, Fout), jnp.float32),
    )(p0, p1, b, *flat)


_MESH = functools.partial(plsc.VectorSubcoreMesh,
                          core_axis_name="c", subcore_axis_name="s",
                          num_cores=NC, num_subcores=NS)
_SC_PARAMS = functools.partial(pltpu.CompilerParams,
                               use_tc_tiling_on_sc=False)


def _sc_pass1(es, ed, mes, med, srcp, dstp):
    @functools.partial(
        pl.kernel,
        out_type=[jax.ShapeDtypeStruct((EP, 16), jnp.float32),
                  jax.ShapeDtypeStruct((NC, NR, 16), jnp.float32)],
        mesh=_MESH(),
        compiler_params=_SC_PARAMS(),
        scratch_types=[
            pltpu.VMEM((CH,), jnp.int32),
            pltpu.VMEM((CH,), jnp.int32),
            pltpu.VMEM((CH, 16), jnp.float32),
            pltpu.VMEM((CH, 16), jnp.float32),
            pltpu.VMEM((CH, 16), jnp.float32),
            pltpu.VMEM((8, 16), jnp.float32),
            pltpu.VMEM((8, 16), jnp.float32),
            pltpu.VMEM((RPW, 16), jnp.float32),
            pltpu.VMEM_SHARED((NR, 16), jnp.float32),
            pltpu.VMEM_SHARED((NR, 16), jnp.float32),
            pltpu.VMEM_SHARED((NR, 16), jnp.float32),
            pltpu.SemaphoreType.DMA,
        ],
    )
    def k(es_hbm, ed_hbm, mes_hbm, med_hbm, src_hbm, dst_hbm,
          ee_hbm, denp_hbm,
          src_v, dst_v, esg, edg, eev, mbufa, mbufb, zbuf,
          den_sh, es_sh, ed_sh, sem):
        cid = lax.axis_index("c")
        sid = lax.axis_index("s")
        wid = sid * NC + cid
        r0 = sid * RPW

        pltpu.sync_copy(mes_hbm, mbufa)
        pltpu.sync_copy(med_hbm, mbufb)
        msum = mbufa[0] + mbufb[0]
        mr = jnp.maximum(msum, 0.2 * msum)

        pltpu.sync_copy(es_hbm.at[pl.ds(r0, RPW)], zbuf)
        pltpu.sync_copy(zbuf, es_sh.at[pl.ds(r0, RPW)])
        pltpu.sync_copy(ed_hbm.at[pl.ds(r0, RPW)], zbuf)
        pltpu.sync_copy(zbuf, ed_sh.at[pl.ds(r0, RPW)])

        z16 = jnp.zeros((16,), jnp.float32)

        def zb(i, c):
            zbuf[i] = z16
            return c

        lax.fori_loop(0, RPW, zb, 0)
        pltpu.sync_copy(zbuf, den_sh.at[pl.ds(r0, RPW)])
        plsc.subcore_barrier()

        def chunk(kc, m):
            base = (wid * NCHUNK + kc) * CH
            pltpu.sync_copy(src_hbm.at[pl.ds(base, CH)], src_v)
            pltpu.sync_copy(dst_hbm.at[pl.ds(base, CH)], dst_v)
            pltpu.async_copy(es_sh.at[src_v], esg, sem).wait()
            pltpu.async_copy(ed_sh.at[dst_v], edg, sem).wait()

            def eb(i, mm):
                e = esg[i] + edg[i]
                e = jnp.maximum(e, 0.2 * e)
                eev[i] = jnp.exp(e - mm)
                return mm

            m = lax.fori_loop(0, CH, eb, m)
            pltpu.sync_copy(eev, ee_hbm.at[pl.ds(base, CH)])
            pltpu.sync_copy(eev, den_sh.at[dst_v], add=True)
            return m

        lax.fori_loop(0, NCHUNK, chunk, mr)
        plsc.subcore_barrier()
        pltpu.sync_copy(den_sh.at[pl.ds(r0, RPW)],
                        denp_hbm.at[cid, pl.ds(r0, RPW)])

    return k(es, ed, mes, med, srcp, dstp)


def _sc_pass2(h, rden, ee, srcp, dstp, F, CHN, j0):
    NV = F // 16
    H0 = [(16 * (j0 + j)) // CHN for j in range(NV)]
    TH = [(H0[j] + 1) * CHN - 16 * (j0 + j) for j in range(NV)]
    H1 = [min(H0[j] + 1, 15) for j in range(NV)]

    @functools.partial(
        pl.kernel,
        out_type=jax.ShapeDtypeStruct((NC, NR, F), jnp.float32),
        mesh=_MESH(),
        compiler_params=_SC_PARAMS(),
        scratch_types=[
            [pltpu.VMEM((CH,), jnp.int32)] * 2,
            [pltpu.VMEM((CH,), jnp.int32)] * 2,
            [pltpu.VMEM((CH, 16), jnp.float32)] * 2,
            [pltpu.VMEM((CH, 128), jnp.float32)] * 2,
            [pltpu.VMEM((CH, 128), jnp.float32)] * 2,
            pltpu.VMEM((CH, F), jnp.float32),
            pltpu.VMEM((ZB, F), jnp.float32),
            pltpu.VMEM_SHARED((NR, F), jnp.float32),
            [pltpu.SemaphoreType.DMA] * 2,
            [pltpu.SemaphoreType.DMA] * 2,
            [pltpu.SemaphoreType.DMA] * 2,
        ],
    )
    def k(h_hbm, rden_hbm, ee_hbm, src_hbm, dst_hbm, outp_hbm,
          src_v, dst_v, eev, rdg, hg, msg, zbuf,
          out_sh, semi, semd, semh):
        cid = lax.axis_index("c")
        sid = lax.axis_index("s")
        wid = sid * NC + cid
        r0 = sid * RPW

        z16 = jnp.zeros((16,), jnp.float32)

        def zb(i, c):
            for j in range(NV):
                zbuf[i, pl.ds(16 * j, 16)] = z16
            return c

        lax.fori_loop(0, ZB, zb, 0)
        for q in range(RPW // ZB):
            pltpu.sync_copy(zbuf, out_sh.at[pl.ds(r0 + q * ZB, ZB)])
        plsc.subcore_barrier()

        lane = lax.iota(jnp.int32, 16)
        masks = [lane < TH[j] for j in range(NV)]
        base0 = wid * NCHUNK * CH

        def issue_ie(kc, b):
            base = base0 + kc * CH
            pltpu.async_copy(src_hbm.at[pl.ds(base, CH)], src_v[b], semi[b])
            pltpu.async_copy(dst_hbm.at[pl.ds(base, CH)], dst_v[b], semi[b])
            pltpu.async_copy(ee_hbm.at[pl.ds(base, CH)], eev[b], semi[b])

        def wait_ie(kc, b):
            base = base0 + kc * CH
            pltpu.make_async_copy(src_hbm.at[pl.ds(base, CH)], src_v[b],
                                  semi[b]).wait()
            pltpu.make_async_copy(dst_hbm.at[pl.ds(base, CH)], dst_v[b],
                                  semi[b]).wait()
            pltpu.make_async_copy(ee_hbm.at[pl.ds(base, CH)], eev[b],
                                  semi[b]).wait()

        def issue_g(b):
            pltpu.async_copy(rden_hbm.at[dst_v[b]], rdg[b], semd[b])
            pltpu.async_copy(h_hbm.at[src_v[b]], hg[b], semh[b])

        def wait_g(b):
            pltpu.make_async_copy(rden_hbm.at[dst_v[b]], rdg[b],
                                  semd[b]).wait()
            pltpu.make_async_copy(h_hbm.at[src_v[b]], hg[b], semh[b]).wait()

        def compute(b):
            def mb(i, cc):
                av = eev[b][i]
                for j in range(NV):
                    e0 = av[H0[j]]
                    e1 = av[H1[j]]
                    eexp = jnp.where(masks[j], e0, e1)
                    msg[i, pl.ds(16 * j, 16)] = (
                        hg[b][i, pl.ds(16 * j, 16)]
                        * rdg[b][i, pl.ds(16 * j, 16)] * eexp)
                return cc

            lax.fori_loop(0, CH, mb, 0)
            pltpu.sync_copy(msg, out_sh.at[dst_v[b]], add=True)

        issue_ie(0, 0)
        wait_ie(0, 0)
        issue_g(0)
        issue_ie(1, 1)

        @pl.loop(0, NCHUNK - 2, step=2)
        def _(kc):
            for b in range(2):
                kk = kc + b
                wait_ie(kk + 1, 1 - b)
                issue_g(1 - b)
                wait_g(b)
                compute(b)
                issue_ie(kk + 2, b)

        for kk, b in ((NCHUNK - 2, 0), (NCHUNK - 1, 1)):
            if kk == NCHUNK - 2:
                wait_ie(kk + 1, 1 - b)
                issue_g(1 - b)
            wait_g(b)
            compute(b)

        plsc.subcore_barrier()
        pltpu.sync_copy(out_sh.at[pl.ds(r0, RPW)],
                        outp_hbm.at[cid, pl.ds(r0, RPW)])

    return k(h, rden, ee, srcp, dstp)


def _attn_mat(a, H, C):
    A = jnp.zeros((H * C, 16), jnp.float32)
    return A.at[jnp.arange(H * C), jnp.repeat(jnp.arange(H), C)].set(
        a.reshape(-1))


def _bn_layer(W, b, g, bt):
    c = 1.0 / jnp.sqrt(jnp.float32(1.0 + 1e-5))
    return (W, b[None], (g * c)[None], bt[None])


def _pad_cols(a, n):
    return jnp.pad(a, ((0, 0), (0, n - a.shape[1])))


def _pad_rows(a, n):
    return jnp.pad(a, ((0, n - a.shape[0]), (0, 0)))


def _erep(CHN, col0, width):
    c = jnp.arange(128)
    hidx = (col0 + c) // CHN
    valid = c < width
    Emat = (hidx[None, :] == jnp.arange(16)[:, None]) & valid[None, :]
    return Emat.astype(jnp.float32)


def kernel(x, edge_index, params):
    src = edge_index[0]
    dst = edge_index[1]
    loop = jnp.arange(N, dtype=src.dtype)
    padi = jnp.full((EP - EL,), N, src.dtype)
    srcp = jnp.concatenate([src, loop, padi])
    dstp = jnp.concatenate([dst, loop, padi])
    xp = jnp.pad(x, ((0, NR - N), (0, 0)))

    W1, as1, ad1, b1 = params['conv1']
    W4, as4, ad4, b4 = params['conv4']
    As1 = _attn_mat(as1, 12, 12)
    Ad1 = _attn_mat(ad1, 12, 12)
    W1ks = [_pad_cols(W1[:, 48 * k:48 * k + 48], 128) for k in range(3)]
    As1ks = [_pad_rows(As1[48 * k:48 * k + 48], 128) for k in range(3)]
    Ad1ks = [_pad_rows(Ad1[48 * k:48 * k + 48], 128) for k in range(3)]
    b1ks = [b1[None, 48 * k:48 * k + 48] for k in range(3)]
    W4ks = [_pad_cols(W4[48 * k:48 * k + 48], 128) for k in range(3)]
    As4 = _pad_rows(_attn_mat(as4, 8, 8), 128)
    Ad4 = _pad_rows(_attn_mat(ad4, 8, 8), 128)

    layers = [_bn_layer(*lay) for lay in params['mlp']]
    heads = params['heads']
    for li in range(3):
        Ws = [h[li][0] for h in heads]
        bs = jnp.concatenate([h[li][1] for h in heads])
        gs = jnp.concatenate([h[li][2] for h in heads])
        bts = jnp.concatenate([h[li][3] for h in heads])
        Wcat = jax.scipy.linalg.block_diag(*Ws) if li > 0 else \
            jnp.concatenate(Ws, axis=1)
        if li == 2:
            Wcat = jnp.pad(Wcat, ((0, 0), (0, 5)))
            bs = jnp.pad(bs, (0, 5))
            gs = jnp.pad(gs, (0, 5))
            bts = jnp.pad(bts, (0, 5))
        layers.append(_bn_layer(Wcat, bs, gs, bts))

    h1a, h1b, h1c, es1, ed1, mes1, med1 = _tc_stage1(xp, W1ks, As1ks, Ad1ks)
    ee1, denp1 = _sc_pass1(es1, ed1, mes1, med1, srcp, dstp)
    rden1s = _tc_rden(denp1, [_erep(12, 48 * k, 48) for k in range(3)])
    outs1 = [_sc_pass2(hk, rden1s[k], ee1, srcp, dstp, 48, 12, 3 * k)
             for k, hk in enumerate((h1a, h1b, h1c))]
    h2, es2, ed2, mes2, med2 = _tc_stage2(
        [o[0] for o in outs1], [o[1] for o in outs1], b1ks, W4ks, As4, Ad4)
    ee2, denp2 = _sc_pass1(es2, ed2, mes2, med2, srcp, dstp)
    rden2 = _tc_rden(denp2, [_erep(8, 0, 64)])[0]
    outp2 = _sc_pass2(h2, rden2, ee2, srcp, dstp, 64, 8, 0)
    out = _tc_stage3(outp2[0], outp2[1], b4[None], layers)
    return out[:N, :3]


# speedup vs baseline: 32.9461x; 1.0008x over previous
import functools

import jax
import jax.numpy as jnp
from jax import lax
from jax.experimental import pallas as pl
from jax.experimental.pallas import tpu as pltpu
from jax.experimental.pallas import tpu_sc as plsc

N = 10000
FEA = 128
E = 320000
EL = E + N
NC, NS = 2, 16
NW = NC * NS
CH = 128
NCHUNK = -(-EL // (NW * CH)) + 1
EP = NW * CH * NCHUNK
RPW = 640
NR = RPW * NS
ZB = RPW // 4
SPLITS1 = ((0, 48), (48, 48), (96, 48))

_PREC = jax.lax.Precision.HIGHEST
_NEG = -3e38


def _dot(a, b):
    return jax.lax.dot(a, b, precision=_PREC,
                       preferred_element_type=jnp.float32)


def _elu(z):
    return jnp.where(z > 0, z, jnp.exp(jnp.minimum(z, 0.0)) - 1.0)


def _tc1_body(nk, *refs):
    x_ref = refs[0]
    w_refs = refs[1:1 + nk]
    as_refs = refs[1 + nk:1 + 2 * nk]
    ad_refs = refs[1 + 2 * nk:1 + 3 * nk]
    h_refs = refs[1 + 3 * nk:1 + 4 * nk]
    es_ref, ed_ref, mes_ref, med_ref = refs[1 + 4 * nk:]
    i = pl.program_id(0)
    x = x_ref[...]
    es = jnp.zeros((x.shape[0], 16), jnp.float32)
    ed = jnp.zeros((x.shape[0], 16), jnp.float32)
    for k in range(nk):
        h = _dot(x, w_refs[k][...])
        h_refs[k][...] = h
        es = es + _dot(h, as_refs[k][...])
        ed = ed + _dot(h, ad_refs[k][...])
    es_ref[...] = es
    ed_ref[...] = ed

    @pl.when(i == 0)
    def _():
        mes_ref[...] = jnp.full((8, 16), _NEG, jnp.float32)
        med_ref[...] = jnp.full((8, 16), _NEG, jnp.float32)

    mes_ref[...] = jnp.maximum(mes_ref[...],
                               jnp.broadcast_to(jnp.max(es, axis=0), (8, 16)))
    med_ref[...] = jnp.maximum(med_ref[...],
                               jnp.broadcast_to(jnp.max(ed, axis=0), (8, 16)))


def _tc_stage1(xp, Wks, Asks, Adks):
    BLK = 2560
    G = NR // BLK
    const = lambda a: pl.BlockSpec(a.shape, lambda i: (0, 0))
    row = lambda w: pl.BlockSpec((BLK, w), lambda i: (i, 0))
    nk = len(Wks)
    return pl.pallas_call(
        functools.partial(_tc1_body, nk),
        grid=(G,),
        in_specs=[row(FEA)] + [const(a) for a in Wks + Asks + Adks],
        out_specs=[row(128)] * nk + [row(16), row(16),
                   const(jnp.zeros((8, 16))), const(jnp.zeros((8, 16)))],
        out_shape=[jax.ShapeDtypeStruct((NR, 128), jnp.float32)] * nk +
                  [jax.ShapeDtypeStruct((NR, 16), jnp.float32)] * 2 +
                  [jax.ShapeDtypeStruct((8, 16), jnp.float32)] * 2,
    )(xp, *Wks, *Asks, *Adks)


def _tc2_body(nk, *refs):
    p0s = refs[0:nk]
    p1s = refs[nk:2 * nk]
    bs = refs[2 * nk:3 * nk]
    w4s = refs[3 * nk:4 * nk]
    as_ref, ad_ref = refs[4 * nk:4 * nk + 2]
    h_ref, es_ref, ed_ref, mes_ref, med_ref = refs[4 * nk + 2:]
    i = pl.program_id(0)
    h = jnp.zeros((p0s[0].shape[0], 128), jnp.float32)
    for k in range(nk):
        z = _elu(p0s[k][...] + p1s[k][...] + bs[k][...])
        h = h + _dot(z, w4s[k][...])
    h_ref[...] = h
    es = _dot(h, as_ref[...])
    ed = _dot(h, ad_ref[...])
    es_ref[...] = es
    ed_ref[...] = ed

    @pl.when(i == 0)
    def _():
        mes_ref[...] = jnp.full((8, 16), _NEG, jnp.float32)
        med_ref[...] = jnp.full((8, 16), _NEG, jnp.float32)

    mes_ref[...] = jnp.maximum(mes_ref[...],
                               jnp.broadcast_to(jnp.max(es, axis=0), (8, 16)))
    med_ref[...] = jnp.maximum(med_ref[...],
                               jnp.broadcast_to(jnp.max(ed, axis=0), (8, 16)))


def _tc_stage2(p0s, p1s, bks, W4ks, As4, Ad4):
    BLK = 2560
    G = NR // BLK
    const = lambda a: pl.BlockSpec(a.shape, lambda i: (0, 0))
    row = lambda w: pl.BlockSpec((BLK, w), lambda i: (i, 0))
    widths = [b.shape[1] for b in bks]
    in_specs = ([row(w) for w in widths] * 2 + [const(b) for b in bks]
                + [const(w) for w in W4ks] + [const(As4), const(Ad4)])
    return pl.pallas_call(
        functools.partial(_tc2_body, len(bks)),
        grid=(G,),
        in_specs=in_specs,
        out_specs=[row(128), row(16), row(16),
                   const(jnp.zeros((8, 16))), const(jnp.zeros((8, 16)))],
        out_shape=[jax.ShapeDtypeStruct((NR, 128), jnp.float32),
                   jax.ShapeDtypeStruct((NR, 16), jnp.float32),
                   jax.ShapeDtypeStruct((NR, 16), jnp.float32),
                   jax.ShapeDtypeStruct((8, 16), jnp.float32),
                   jax.ShapeDtypeStruct((8, 16), jnp.float32)],
    )(*p0s, *p1s, *bks, *W4ks, As4, Ad4)


def _tcr_body(*refs):
    p0_ref, p1_ref = refs[0], refs[1]
    ne = (len(refs) - 2) // 2
    e_refs = refs[2:2 + ne]
    out_refs = refs[2 + ne:]
    rden = 1.0 / (p0_ref[...] + p1_ref[...] + 1e-16)
    for er, outr in zip(e_refs, out_refs):
        outr[...] = _dot(rden, er[...])


def _tc_rden(denp, Ereps):
    BLK = 2560
    G = NR // BLK
    const = lambda a: pl.BlockSpec(a.shape, lambda i: (0, 0))
    row = lambda w: pl.BlockSpec((BLK, w), lambda i: (i, 0))
    ne = len(Ereps)
    return pl.pallas_call(
        _tcr_body,
        grid=(G,),
        in_specs=[row(16), row(16)] + [const(e) for e in Ereps],
        out_specs=[row(128)] * ne,
        out_shape=[jax.ShapeDtypeStruct((NR, 128), jnp.float32)] * ne,
    )(denp[0], denp[1], *Ereps)


def _tc3_body(*refs):
    p0_ref, p1_ref, b_ref = refs[0], refs[1], refs[2]
    w_refs = refs[3:-1]
    out_ref = refs[-1]
    z = _elu(p0_ref[...] + p1_ref[...] + b_ref[...])
    for li in range(len(w_refs) // 4):
        W, b, s, bt = w_refs[4 * li:4 * li + 4]
        z = jnp.maximum(_dot(z, W[...]) + b[...], 0.0) * s[...] + bt[...]
    out_ref[...] = z


def _tc_stage3(p0, p1, b, layers):
    BLK = 2560
    G = NR // BLK
    Fin = p0.shape[1]
    flat = []
    for lay in layers:
        flat.extend(lay)
    in_specs = [pl.BlockSpec((BLK, Fin), lambda i: (i, 0)),
                pl.BlockSpec((BLK, Fin), lambda i: (i, 0)),
                pl.BlockSpec((1, Fin), lambda i: (0, 0))]
    for a in flat:
        in_specs.append(pl.BlockSpec(a.shape, lambda i: (0, 0)))
    Fout = layers[-1][0].shape[1]
    return pl.pallas_call(
        _tc3_body,
        grid=(G,),
        in_specs=in_specs,
        out_specs=pl.BlockSpec((BLK, Fout), lambda i: (i, 0)),
        out_shape=jax.ShapeDtypeStruct((NR, Fout), jnp.float32),
    )(p0, p1, b, *flat)


_MESH = functools.partial(plsc.VectorSubcoreMesh,
                          core_axis_name="c", subcore_axis_name="s",
                          num_cores=NC, num_subcores=NS)
_SC_PARAMS = functools.partial(pltpu.CompilerParams,
                               use_tc_tiling_on_sc=False)


def _sc_pass1(es, ed, mes, med, srcp, dstp):
    @functools.partial(
        pl.kernel,
        out_type=[jax.ShapeDtypeStruct((EP, 16), jnp.float32),
                  jax.ShapeDtypeStruct((NC, NR, 16), jnp.float32)],
        mesh=_MESH(),
        compiler_params=_SC_PARAMS(),
        scratch_types=[
            pltpu.VMEM((CH,), jnp.int32),
            pltpu.VMEM((CH,), jnp.int32),
            pltpu.VMEM((CH, 16), jnp.float32),
            pltpu.VMEM((CH, 16), jnp.float32),
            pltpu.VMEM((CH, 16), jnp.float32),
            pltpu.VMEM((8, 16), jnp.float32),
            pltpu.VMEM((8, 16), jnp.float32),
            pltpu.VMEM((RPW, 16), jnp.float32),
            pltpu.VMEM_SHARED((NR, 16), jnp.float32),
            pltpu.VMEM_SHARED((NR, 16), jnp.float32),
            pltpu.VMEM_SHARED((NR, 16), jnp.float32),
            pltpu.SemaphoreType.DMA,
        ],
    )
    def k(es_hbm, ed_hbm, mes_hbm, med_hbm, src_hbm, dst_hbm,
          ee_hbm, denp_hbm,
          src_v, dst_v, esg, edg, eev, mbufa, mbufb, zbuf,
          den_sh, es_sh, ed_sh, sem):
        cid = lax.axis_index("c")
        sid = lax.axis_index("s")
        wid = sid * NC + cid
        r0 = sid * RPW

        pltpu.sync_copy(mes_hbm, mbufa)
        pltpu.sync_copy(med_hbm, mbufb)
        msum = mbufa[0] + mbufb[0]
        mr = jnp.maximum(msum, 0.2 * msum)

        pltpu.sync_copy(es_hbm.at[pl.ds(r0, RPW)], zbuf)
        pltpu.sync_copy(zbuf, es_sh.at[pl.ds(r0, RPW)])
        pltpu.sync_copy(ed_hbm.at[pl.ds(r0, RPW)], zbuf)
        pltpu.sync_copy(zbuf, ed_sh.at[pl.ds(r0, RPW)])

        z16 = jnp.zeros((16,), jnp.float32)

        def zb(i, c):
            zbuf[i] = z16
            return c

        lax.fori_loop(0, RPW, zb, 0)
        pltpu.sync_copy(zbuf, den_sh.at[pl.ds(r0, RPW)])
        plsc.subcore_barrier()

        def chunk(kc, m):
            base = (wid * NCHUNK + kc) * CH
            pltpu.sync_copy(src_hbm.at[pl.ds(base, CH)], src_v)
            pltpu.sync_copy(dst_hbm.at[pl.ds(base, CH)], dst_v)
            pltpu.async_copy(es_sh.at[src_v], esg, sem).wait()
            pltpu.async_copy(ed_sh.at[dst_v], edg, sem).wait()

            def eb(i, mm):
                e = esg[i] + edg[i]
                e = jnp.maximum(e, 0.2 * e)
                eev[i] = jnp.exp(e - mm)
                return mm

            m = lax.fori_loop(0, CH, eb, m)
            pltpu.sync_copy(eev, ee_hbm.at[pl.ds(base, CH)])
            pltpu.sync_copy(eev, den_sh.at[dst_v], add=True)
            return m

        lax.fori_loop(0, NCHUNK, chunk, mr)
        plsc.subcore_barrier()
        pltpu.sync_copy(den_sh.at[pl.ds(r0, RPW)],
                        denp_hbm.at[cid, pl.ds(r0, RPW)])

    return k(es, ed, mes, med, srcp, dstp)


def _sc_pass2(h, rden, ee, srcp, dstp, F, CHN, j0):
    NV = F // 16
    H0 = [(16 * (j0 + j)) // CHN for j in range(NV)]
    TH = [(H0[j] + 1) * CHN - 16 * (j0 + j) for j in range(NV)]
    H1 = [min(H0[j] + 1, 15) for j in range(NV)]

    @functools.partial(
        pl.kernel,
        out_type=jax.ShapeDtypeStruct((NC, NR, F), jnp.float32),
        mesh=_MESH(),
        compiler_params=_SC_PARAMS(),
        scratch_types=[
            [pltpu.VMEM((CH,), jnp.int32)] * 2,
            [pltpu.VMEM((CH,), jnp.int32)] * 2,
            [pltpu.VMEM((CH, 16), jnp.float32)] * 2,
            [pltpu.VMEM((CH, 128), jnp.float32)] * 2,
            [pltpu.VMEM((CH, 128), jnp.float32)] * 2,
            pltpu.VMEM((CH, F), jnp.float32),
            pltpu.VMEM((ZB, F), jnp.float32),
            pltpu.VMEM_SHARED((NR, F), jnp.float32),
            [pltpu.SemaphoreType.DMA] * 2,
            [pltpu.SemaphoreType.DMA] * 2,
            [pltpu.SemaphoreType.DMA] * 2,
        ],
    )
    def k(h_hbm, rden_hbm, ee_hbm, src_hbm, dst_hbm, outp_hbm,
          src_v, dst_v, eev, rdg, hg, msg, zbuf,
          out_sh, semi, semd, semh):
        cid = lax.axis_index("c")
        sid = lax.axis_index("s")
        wid = sid * NC + cid
        r0 = sid * RPW

        z16 = jnp.zeros((16,), jnp.float32)

        def zb(i, c):
            for j in range(NV):
                zbuf[i, pl.ds(16 * j, 16)] = z16
            return c

        lax.fori_loop(0, ZB, zb, 0)
        for q in range(RPW // ZB):
            pltpu.sync_copy(zbuf, out_sh.at[pl.ds(r0 + q * ZB, ZB)])
        plsc.subcore_barrier()

        lane = lax.iota(jnp.int32, 16)
        masks = [lane < TH[j] for j in range(NV)]
        base0 = wid * NCHUNK * CH

        def issue_ie(kc, b):
            base = base0 + kc * CH
            pltpu.async_copy(src_hbm.at[pl.ds(base, CH)], src_v[b], semi[b])
            pltpu.async_copy(dst_hbm.at[pl.ds(base, CH)], dst_v[b], semi[b])
            pltpu.async_copy(ee_hbm.at[pl.ds(base, CH)], eev[b], semi[b])

        def wait_ie(kc, b):
            base = base0 + kc * CH
            pltpu.make_async_copy(src_hbm.at[pl.ds(base, CH)], src_v[b],
                                  semi[b]).wait()
            pltpu.make_async_copy(dst_hbm.at[pl.ds(base, CH)], dst_v[b],
                                  semi[b]).wait()
            pltpu.make_async_copy(ee_hbm.at[pl.ds(base, CH)], eev[b],
                                  semi[b]).wait()

        def issue_g(b):
            pltpu.async_copy(rden_hbm.at[dst_v[b]], rdg[b], semd[b])
            pltpu.async_copy(h_hbm.at[src_v[b]], hg[b], semh[b])

        def wait_g(b):
            pltpu.make_async_copy(rden_hbm.at[dst_v[b]], rdg[b],
                                  semd[b]).wait()
            pltpu.make_async_copy(h_hbm.at[src_v[b]], hg[b], semh[b]).wait()

        def compute(b):
            def mb(i, cc):
                av = eev[b][i]
                for j in range(NV):
                    e0 = av[H0[j]]
                    e1 = av[H1[j]]
                    eexp = jnp.where(masks[j], e0, e1)
                    msg[i, pl.ds(16 * j, 16)] = (
                        hg[b][i, pl.ds(16 * j, 16)]
                        * rdg[b][i, pl.ds(16 * j, 16)] * eexp)
                return cc

            lax.fori_loop(0, CH, mb, 0, unroll=4)
            pltpu.sync_copy(msg, out_sh.at[dst_v[b]], add=True)

        issue_ie(0, 0)
        wait_ie(0, 0)
        issue_g(0)
        issue_ie(1, 1)

        @pl.loop(0, NCHUNK - 2, step=2)
        def _(kc):
            for b in range(2):
                kk = kc + b
                wait_ie(kk + 1, 1 - b)
                issue_g(1 - b)
                wait_g(b)
                compute(b)
                issue_ie(kk + 2, b)

        for kk, b in ((NCHUNK - 2, 0), (NCHUNK - 1, 1)):
            if kk == NCHUNK - 2:
                wait_ie(kk + 1, 1 - b)
                issue_g(1 - b)
            wait_g(b)
            compute(b)

        plsc.subcore_barrier()
        pltpu.sync_copy(out_sh.at[pl.ds(r0, RPW)],
                        outp_hbm.at[cid, pl.ds(r0, RPW)])

    return k(h, rden, ee, srcp, dstp)


def _attn_mat(a, H, C):
    A = jnp.zeros((H * C, 16), jnp.float32)
    return A.at[jnp.arange(H * C), jnp.repeat(jnp.arange(H), C)].set(
        a.reshape(-1))


def _bn_layer(W, b, g, bt):
    c = 1.0 / jnp.sqrt(jnp.float32(1.0 + 1e-5))
    return (W, b[None], (g * c)[None], bt[None])


def _pad_cols(a, n):
    return jnp.pad(a, ((0, 0), (0, n - a.shape[1])))


def _pad_rows(a, n):
    return jnp.pad(a, ((0, n - a.shape[0]), (0, 0)))


def _erep(CHN, col0, width):
    c = jnp.arange(128)
    hidx = (col0 + c) // CHN
    valid = c < width
    Emat = (hidx[None, :] == jnp.arange(16)[:, None]) & valid[None, :]
    return Emat.astype(jnp.float32)


def kernel(x, edge_index, params):
    src = edge_index[0]
    dst = edge_index[1]
    loop = jnp.arange(N, dtype=src.dtype)
    padi = jnp.full((EP - EL,), N, src.dtype)
    srcp = jnp.concatenate([src, loop, padi])
    dstp = jnp.concatenate([dst, loop, padi])
    xp = jnp.pad(x, ((0, NR - N), (0, 0)))

    W1, as1, ad1, b1 = params['conv1']
    W4, as4, ad4, b4 = params['conv4']
    As1 = _attn_mat(as1, 12, 12)
    Ad1 = _attn_mat(ad1, 12, 12)
    W1ks = [_pad_cols(W1[:, c0:c0 + w], 128) for c0, w in SPLITS1]
    As1ks = [_pad_rows(As1[c0:c0 + w], 128) for c0, w in SPLITS1]
    Ad1ks = [_pad_rows(Ad1[c0:c0 + w], 128) for c0, w in SPLITS1]
    b1ks = [b1[None, c0:c0 + w] for c0, w in SPLITS1]
    W4ks = [_pad_cols(W4[c0:c0 + w], 128) for c0, w in SPLITS1]
    As4 = _pad_rows(_attn_mat(as4, 8, 8), 128)
    Ad4 = _pad_rows(_attn_mat(ad4, 8, 8), 128)

    layers = [_bn_layer(*lay) for lay in params['mlp']]
    heads = params['heads']
    for li in range(3):
        Ws = [h[li][0] for h in heads]
        bs = jnp.concatenate([h[li][1] for h in heads])
        gs = jnp.concatenate([h[li][2] for h in heads])
        bts = jnp.concatenate([h[li][3] for h in heads])
        Wcat = jax.scipy.linalg.block_diag(*Ws) if li > 0 else \
            jnp.concatenate(Ws, axis=1)
        if li == 2:
            Wcat = jnp.pad(Wcat, ((0, 0), (0, 5)))
            bs = jnp.pad(bs, (0, 5))
            gs = jnp.pad(gs, (0, 5))
            bts = jnp.pad(bts, (0, 5))
        layers.append(_bn_layer(Wcat, bs, gs, bts))

    res1 = _tc_stage1(xp, W1ks, As1ks, Ad1ks)
    h1ks = res1[:len(SPLITS1)]
    es1, ed1, mes1, med1 = res1[len(SPLITS1):]
    ee1, denp1 = _sc_pass1(es1, ed1, mes1, med1, srcp, dstp)
    rden1s = _tc_rden(denp1, [_erep(12, c0, w) for c0, w in SPLITS1])
    outs1 = [_sc_pass2(hk, rden1s[k], ee1, srcp, dstp, w, 12, c0 // 16)
             for k, (hk, (c0, w)) in enumerate(zip(h1ks, SPLITS1))]
    h2, es2, ed2, mes2, med2 = _tc_stage2(
        [o[0] for o in outs1], [o[1] for o in outs1], b1ks, W4ks, As4, Ad4)
    ee2, denp2 = _sc_pass1(es2, ed2, mes2, med2, srcp, dstp)
    rden2 = _tc_rden(denp2, [_erep(8, 0, 64)])[0]
    outp2 = _sc_pass2(h2, rden2, ee2, srcp, dstp, 64, 8, 0)
    out = _tc_stage3(outp2[0], outp2[1], b4[None], layers)
    return out[:N, :3]
